# baseline (device time: 109565 ns/iter reference)
import jax
import jax.numpy as jnp
from jax import lax
from jax.experimental import pallas as pl
from jax.experimental.pallas import tpu as pltpu


def kernel(x, assign, W1, W2):
    T, D = x.shape
    E_LOC, _, F = W1.shape
    FC = 512
    NF = F // FC

    assign2 = assign.reshape(T, 1)

    def gather_body(x_ref, a_ref, xall_ref, aall_ref, send_sems, recv_sems):
        my_x = lax.axis_index("x")
        my_y = lax.axis_index("y")
        peer_y = 1 - my_y
        start = my_y * T

        xall_ref[pl.ds(start, T), :] = x_ref[...].astype(jnp.bfloat16)
        aall_ref[pl.ds(start, T), :] = a_ref[...]

        barrier_sem = pltpu.get_barrier_semaphore()
        pl.semaphore_signal(
            barrier_sem, inc=1,
            device_id=(my_x, peer_y), device_id_type=pl.DeviceIdType.MESH,
        )
        pl.semaphore_wait(barrier_sem, 1)

        rdma_x = pltpu.make_async_remote_copy(
            src_ref=xall_ref.at[pl.ds(start, T), :],
            dst_ref=xall_ref.at[pl.ds(start, T), :],
            send_sem=send_sems.at[0],
            recv_sem=recv_sems.at[0],
            device_id=(my_x, peer_y),
            device_id_type=pl.DeviceIdType.MESH,
        )
        rdma_a = pltpu.make_async_remote_copy(
            src_ref=aall_ref.at[pl.ds(start, T), :],
            dst_ref=aall_ref.at[pl.ds(start, T), :],
            send_sem=send_sems.at[1],
            recv_sem=recv_sems.at[1],
            device_id=(my_x, peer_y),
            device_id_type=pl.DeviceIdType.MESH,
        )
        rdma_x.start()
        rdma_a.start()
        rdma_x.wait()
        rdma_a.wait()

    x_all, a_all = pl.pallas_call(
        gather_body,
        out_shape=[
            jax.ShapeDtypeStruct((2 * T, D), jnp.bfloat16),
            jax.ShapeDtypeStruct((2 * T, 1), jnp.int32),
        ],
        in_specs=[
            pl.BlockSpec(memory_space=pltpu.VMEM),
            pl.BlockSpec(memory_space=pltpu.VMEM),
        ],
        out_specs=[
            pl.BlockSpec(memory_space=pltpu.VMEM),
            pl.BlockSpec(memory_space=pltpu.VMEM),
        ],
        scratch_shapes=[
            pltpu.SemaphoreType.DMA((2,)),
            pltpu.SemaphoreType.DMA((2,)),
        ],
        compiler_params=pltpu.CompilerParams(collective_id=0),
    )(x, assign2)

    def ffn_body(xall_ref, aall_ref, w1_ref, w2_ref, acc_ref, xm_ref):
        le = pl.program_id(0)
        fi = pl.program_id(1)
        my_y = lax.axis_index("y")
        e = my_y * E_LOC + le

        @pl.when(fi == 0)
        def _():
            mask = (aall_ref[...] == e).astype(jnp.bfloat16)
            xm_ref[...] = xall_ref[...] * mask

        w1 = w1_ref[0].astype(jnp.bfloat16)
        h = jnp.maximum(
            jnp.dot(xm_ref[...], w1, preferred_element_type=jnp.float32), 0.0
        ).astype(jnp.bfloat16)
        o = jnp.dot(
            h, w2_ref[0].astype(jnp.bfloat16), preferred_element_type=jnp.float32
        )

        first = jnp.logical_and(le == 0, fi == 0)

        @pl.when(first)
        def _():
            acc_ref[...] = o

        @pl.when(jnp.logical_not(first))
        def _():
            acc_ref[...] += o

    acc = pl.pallas_call(
        ffn_body,
        grid=(E_LOC, NF),
        out_shape=jax.ShapeDtypeStruct((2 * T, D), jnp.float32),
        in_specs=[
            pl.BlockSpec((2 * T, D), lambda le, fi: (0, 0)),
            pl.BlockSpec((2 * T, 1), lambda le, fi: (0, 0)),
            pl.BlockSpec((1, D, FC), lambda le, fi: (le, 0, fi)),
            pl.BlockSpec((1, FC, D), lambda le, fi: (le, fi, 0)),
        ],
        out_specs=pl.BlockSpec((2 * T, D), lambda le, fi: (0, 0)),
        scratch_shapes=[pltpu.VMEM((2 * T, D), jnp.bfloat16)],
    )(x_all, a_all, W1, W2)

    def combine_body(acc_ref, out_ref, send_ref, recv_ref, send_sem, recv_sem):
        my_x = lax.axis_index("x")
        my_y = lax.axis_index("y")
        peer_y = 1 - my_y

        send_ref[...] = acc_ref[pl.ds(peer_y * T, T), :].astype(jnp.bfloat16)

        barrier_sem = pltpu.get_barrier_semaphore()
        pl.semaphore_signal(
            barrier_sem, inc=1,
            device_id=(my_x, peer_y), device_id_type=pl.DeviceIdType.MESH,
        )
        pl.semaphore_wait(barrier_sem, 1)

        rdma = pltpu.make_async_remote_copy(
            src_ref=send_ref,
            dst_ref=recv_ref,
            send_sem=send_sem,
            recv_sem=recv_sem,
            device_id=(my_x, peer_y),
            device_id_type=pl.DeviceIdType.MESH,
        )
        rdma.start()
        rdma.wait()

        out_ref[...] = acc_ref[pl.ds(my_y * T, T), :] + recv_ref[...].astype(
            jnp.float32
        )

    return pl.pallas_call(
        combine_body,
        out_shape=jax.ShapeDtypeStruct((T, D), jnp.float32),
        in_specs=[pl.BlockSpec(memory_space=pltpu.VMEM)],
        out_specs=pl.BlockSpec(memory_space=pltpu.VMEM),
        scratch_shapes=[
            pltpu.VMEM((T, D), jnp.bfloat16),
            pltpu.VMEM((T, D), jnp.bfloat16),
            pltpu.SemaphoreType.DMA,
            pltpu.SemaphoreType.DMA,
        ],
        compiler_params=pltpu.CompilerParams(collective_id=1),
    )(acc)


# device time: 80102 ns/iter; 1.3678x vs baseline; 1.3678x over previous
import jax
import jax.numpy as jnp
from jax import lax
from jax.experimental import pallas as pl
from jax.experimental.pallas import tpu as pltpu


def kernel(x, assign, W1, W2):
    T, D = x.shape
    E_LOC, _, F = W1.shape
    FC = 512
    NF = F // FC
    NK = E_LOC * NF
    NRC = 2
    RCS = T // NRC

    assign2 = assign.reshape(T, 1)

    def body(
        x_ref, a_ref, w1_any, w2_any, out_ref,
        xsend, xrecv, arecv, w1b, w2b, st1, st2,
        acc_my, acc_pr, xm, sendb, recvb,
        w1_sems, w2_sems, gsend, grecv, csend, crecv,
    ):
        my_x = lax.axis_index("x")
        my_y = lax.axis_index("y")
        peer_y = 1 - my_y

        barrier_sem = pltpu.get_barrier_semaphore()
        pl.semaphore_signal(
            barrier_sem, inc=1,
            device_id=(my_x, peer_y), device_id_type=pl.DeviceIdType.MESH,
        )
        pl.semaphore_wait(barrier_sem, 1)

        xsend[...] = x_ref[...].astype(jnp.bfloat16)

        rdma_x = pltpu.make_async_remote_copy(
            src_ref=xsend, dst_ref=xrecv,
            send_sem=gsend.at[0], recv_sem=grecv.at[0],
            device_id=(my_x, peer_y), device_id_type=pl.DeviceIdType.MESH,
        )
        rdma_a = pltpu.make_async_remote_copy(
            src_ref=a_ref, dst_ref=arecv,
            send_sem=gsend.at[1], recv_sem=grecv.at[1],
            device_id=(my_x, peer_y), device_id_type=pl.DeviceIdType.MESH,
        )
        rdma_x.start()
        rdma_a.start()

        def w_chunk_copy(k, slot):
            le, fi = divmod(k, NF)
            c1 = pltpu.make_async_copy(
                w1_any.at[le, :, pl.ds(fi * FC, FC)], st1.at[slot],
                w1_sems.at[slot],
            )
            c2 = pltpu.make_async_copy(
                w2_any.at[le, pl.ds(fi * FC, FC), :], st2.at[slot],
                w2_sems.at[slot],
            )
            c1.start()
            c2.start()
            return c1, c2

        inflight = {0: w_chunk_copy(0, 0), 1: w_chunk_copy(1, 1)}
        for k in range(NK):
            slot = k % 2
            c1, c2 = inflight.pop(k)
            c1.wait()
            c2.wait()
            w1b[:, pl.ds(k * FC, FC)] = st1[slot].astype(jnp.bfloat16)
            w2b[pl.ds(k * FC, FC), :] = st2[slot].astype(jnp.bfloat16)
            if k + 2 < NK:
                inflight[k + 2] = w_chunk_copy(k + 2, slot)

        acc_my[...] = jnp.zeros((T, D), jnp.float32)

        def local_step(k, _):
            e = my_y * E_LOC + k // NF

            @pl.when(k % NF == 0)
            def _():
                xm[...] = xsend[...] * (a_ref[...] == e).astype(jnp.bfloat16)

            fc = pl.ds(k * FC, FC)
            h = jnp.maximum(
                jnp.dot(
                    xm[...], w1b[:, fc], preferred_element_type=jnp.float32
                ),
                0.0,
            ).astype(jnp.bfloat16)
            acc_my[...] += jnp.dot(
                h, w2b[fc, :], preferred_element_type=jnp.float32
            )
            return ()

        lax.fori_loop(0, NK, local_step, ())

        rdma_x.wait_recv()
        rdma_a.wait_recv()

        combines = []
        for rc in range(NRC):
            rows = pl.ds(rc * RCS, RCS)

            acc_pr[rows, :] = jnp.zeros((RCS, D), jnp.float32)

            def remote_step(k, _, rows=rows):
                e = my_y * E_LOC + k // NF

                @pl.when(k % NF == 0)
                def _():
                    xm[rows, :] = xrecv[rows, :] * (
                        arecv[rows, :] == e
                    ).astype(jnp.bfloat16)

                fc = pl.ds(k * FC, FC)
                h = jnp.maximum(
                    jnp.dot(
                        xm[rows, :], w1b[:, fc],
                        preferred_element_type=jnp.float32,
                    ),
                    0.0,
                ).astype(jnp.bfloat16)
                acc_pr[rows, :] += jnp.dot(
                    h, w2b[fc, :], preferred_element_type=jnp.float32
                )
                return ()

            lax.fori_loop(0, NK, remote_step, ())

            sendb[rows, :] = acc_pr[rows, :].astype(jnp.bfloat16)
            cd = pltpu.make_async_remote_copy(
                src_ref=sendb.at[rows, :],
                dst_ref=recvb.at[rows, :],
                send_sem=csend.at[rc], recv_sem=crecv.at[rc],
                device_id=(my_x, peer_y), device_id_type=pl.DeviceIdType.MESH,
            )
            cd.start()
            combines.append(cd)

        for rc, cd in enumerate(combines):
            rows = pl.ds(rc * RCS, RCS)
            cd.wait_recv()
            out_ref[rows, :] = acc_my[rows, :] + recvb[rows, :].astype(
                jnp.float32
            )
        for cd in combines:
            cd.wait_send()
        rdma_x.wait_send()
        rdma_a.wait_send()

    return pl.pallas_call(
        body,
        out_shape=jax.ShapeDtypeStruct((T, D), jnp.float32),
        in_specs=[
            pl.BlockSpec(memory_space=pltpu.VMEM),
            pl.BlockSpec(memory_space=pltpu.VMEM),
            pl.BlockSpec(memory_space=pl.ANY),
            pl.BlockSpec(memory_space=pl.ANY),
        ],
        out_specs=pl.BlockSpec(memory_space=pltpu.VMEM),
        scratch_shapes=[
            pltpu.VMEM((T, D), jnp.bfloat16),
            pltpu.VMEM((T, D), jnp.bfloat16),
            pltpu.VMEM((T, 1), jnp.int32),
            pltpu.VMEM((D, E_LOC * F), jnp.bfloat16),
            pltpu.VMEM((E_LOC * F, D), jnp.bfloat16),
            pltpu.VMEM((2, D, FC), jnp.float32),
            pltpu.VMEM((2, FC, D), jnp.float32),
            pltpu.VMEM((T, D), jnp.float32),
            pltpu.VMEM((T, D), jnp.float32),
            pltpu.VMEM((T, D), jnp.bfloat16),
            pltpu.VMEM((T, D), jnp.bfloat16),
            pltpu.VMEM((T, D), jnp.bfloat16),
            pltpu.SemaphoreType.DMA((2,)),
            pltpu.SemaphoreType.DMA((2,)),
            pltpu.SemaphoreType.DMA((2,)),
            pltpu.SemaphoreType.DMA((2,)),
            pltpu.SemaphoreType.DMA((NRC,)),
            pltpu.SemaphoreType.DMA((NRC,)),
        ],
        compiler_params=pltpu.CompilerParams(
            collective_id=0, vmem_limit_bytes=60 * 1024 * 1024
        ),
    )(x, assign2, W1, W2)


# device time: 74642 ns/iter; 1.4679x vs baseline; 1.0731x over previous
import os

import jax
import jax.numpy as jnp
from jax import lax
from jax.experimental import pallas as pl
from jax.experimental.pallas import tpu as pltpu

_ABL = os.environ.get("KABL", "full")


def kernel(x, assign, W1, W2):
    T, D = x.shape
    E_LOC, _, F = W1.shape
    FC = 512
    NF = F // FC
    FCL = 256
    NFL = F // FCL
    NKL = E_LOC * NFL
    NSL = 4
    NRC = 2
    RCS = T // NRC

    assign2 = assign.reshape(T, 1)
    comm = _ABL in ("full", "commonly")

    def body(
        x_ref, a_ref, w1_any, w2_any, out_ref,
        xsend, xrecv, arecv, w1b, w2b, st1, st2,
        acc_my, acc_pr, hbuf, sendb, recvb,
        w1_sems, w2_sems, gsend, grecv, csend, crecv,
    ):
        my_x = lax.axis_index("x")
        my_y = lax.axis_index("y")
        peer_y = 1 - my_y

        if comm:
            barrier_sem = pltpu.get_barrier_semaphore()
            pl.semaphore_signal(
                barrier_sem, inc=1,
                device_id=(my_x, peer_y), device_id_type=pl.DeviceIdType.MESH,
            )
            pl.semaphore_wait(barrier_sem, 1)

        xsend[...] = x_ref[...].astype(jnp.bfloat16)

        gathers = []
        if comm:
            rdma_a = pltpu.make_async_remote_copy(
                src_ref=a_ref, dst_ref=arecv,
                send_sem=gsend.at[NRC], recv_sem=grecv.at[NRC],
                device_id=(my_x, peer_y), device_id_type=pl.DeviceIdType.MESH,
            )
            rdma_a.start()
            for g in range(NRC):
                rows = pl.ds(g * RCS, RCS)
                gx = pltpu.make_async_remote_copy(
                    src_ref=xsend.at[rows, :], dst_ref=xrecv.at[rows, :],
                    send_sem=gsend.at[g], recv_sem=grecv.at[g],
                    device_id=(my_x, peer_y),
                    device_id_type=pl.DeviceIdType.MESH,
                )
                gx.start()
                gathers.append(gx)

        def combine_chunk(rc):
            rows = pl.ds(rc * RCS, RCS)
            cd = pltpu.make_async_remote_copy(
                src_ref=sendb.at[rows, :], dst_ref=recvb.at[rows, :],
                send_sem=csend.at[rc], recv_sem=crecv.at[rc],
                device_id=(my_x, peer_y), device_id_type=pl.DeviceIdType.MESH,
            )
            cd.start()
            return cd

        if _ABL == "commonly":
            rdma_a.wait_recv()
            for gx in gathers:
                gx.wait_recv()
            sendb[...] = xrecv[...]
            combines = [combine_chunk(rc) for rc in range(NRC)]
            for cd in combines:
                cd.wait_recv()
            out_ref[...] = x_ref[...] + recvb[...].astype(jnp.float32)
            for cd in combines:
                cd.wait_send()
            for gx in gathers:
                gx.wait_send()
            rdma_a.wait_send()
            return

        def w_chunk_copy(k, slot):
            le, fi = divmod(k, NFL)
            c1 = pltpu.make_async_copy(
                w1_any.at[le, :, pl.ds(fi * FCL, FCL)], st1.at[slot],
                w1_sems.at[slot],
            )
            c2 = pltpu.make_async_copy(
                w2_any.at[le, pl.ds(fi * FCL, FCL), :], st2.at[slot],
                w2_sems.at[slot],
            )
            c1.start()
            c2.start()
            return c1, c2

        inflight = {k: w_chunk_copy(k, k) for k in range(NSL)}

        def load_chunks(k_lo, k_hi):
            for k in range(k_lo, k_hi):
                slot = k % NSL
                c1, c2 = inflight.pop(k)
                c1.wait()
                c2.wait()
                w1b[:, pl.ds(k * FCL, FCL)] = st1[slot].astype(jnp.bfloat16)
                w2b[pl.ds(k * FCL, FCL), :] = st2[slot].astype(jnp.bfloat16)
                if k + NSL < NKL:
                    inflight[k + NSL] = w_chunk_copy(k + NSL, slot)

        def expert_out(src, a_src, rows, n_rows, le):
            def h_step(fi, _):
                fc_w = pl.ds((le * NF + fi) * FC, FC)
                fc_h = pl.ds(fi * FC, FC)
                hbuf[pl.ds(0, n_rows), fc_h] = jnp.maximum(
                    jnp.dot(
                        src[rows, :], w1b[:, fc_w],
                        preferred_element_type=jnp.float32,
                    ),
                    0.0,
                ).astype(jnp.bfloat16)
                return ()

            lax.fori_loop(0, NF, h_step, ())
            o = jnp.dot(
                hbuf[pl.ds(0, n_rows), :],
                w2b[pl.ds(le * F, F), :],
                preferred_element_type=jnp.float32,
            )
            e = my_y * E_LOC + le
            return o * (a_src[rows, :] == e).astype(jnp.float32)

        load_chunks(0, NKL)

        if _ABL == "weights":
            out_ref[...] = x_ref[...]
            return

        if comm:
            rdma_a.wait_recv()
            rsrc, rasrc = xrecv, arecv
        else:
            rsrc, rasrc = xsend, a_ref

        combines = []
        for rc in range(NRC):
            if comm:
                gathers[rc].wait_recv()
            rows = pl.ds(rc * RCS, RCS)
            acc_pr[rows, :] = expert_out(rsrc, rasrc, rows, RCS, 0)
            acc_pr[rows, :] += expert_out(rsrc, rasrc, rows, RCS, 1)
            sendb[rows, :] = acc_pr[rows, :].astype(jnp.bfloat16)
            if comm:
                combines.append(combine_chunk(rc))

        all_rows = pl.ds(0, T)
        acc_my[...] = expert_out(xsend, a_ref, all_rows, T, 0)
        acc_my[...] += expert_out(xsend, a_ref, all_rows, T, 1)

        for rc in range(NRC):
            rows = pl.ds(rc * RCS, RCS)
            if comm:
                combines[rc].wait_recv()
            out_ref[rows, :] = acc_my[rows, :] + recvb[rows, :].astype(
                jnp.float32
            )
        if comm:
            for cd in combines:
                cd.wait_send()
            for gx in gathers:
                gx.wait_send()
            rdma_a.wait_send()

    return pl.pallas_call(
        body,
        out_shape=jax.ShapeDtypeStruct((T, D), jnp.float32),
        in_specs=[
            pl.BlockSpec(memory_space=pltpu.VMEM),
            pl.BlockSpec(memory_space=pltpu.VMEM),
            pl.BlockSpec(memory_space=pl.ANY),
            pl.BlockSpec(memory_space=pl.ANY),
        ],
        out_specs=pl.BlockSpec(memory_space=pltpu.VMEM),
        scratch_shapes=[
            pltpu.VMEM((T, D), jnp.bfloat16),
            pltpu.VMEM((T, D), jnp.bfloat16),
            pltpu.VMEM((T, 1), jnp.int32),
            pltpu.VMEM((D, E_LOC * F), jnp.bfloat16),
            pltpu.VMEM((E_LOC * F, D), jnp.bfloat16),
            pltpu.VMEM((NSL, D, FCL), jnp.float32),
            pltpu.VMEM((NSL, FCL, D), jnp.float32),
            pltpu.VMEM((T, D), jnp.float32),
            pltpu.VMEM((T, D), jnp.float32),
            pltpu.VMEM((T, F), jnp.bfloat16),
            pltpu.VMEM((T, D), jnp.bfloat16),
            pltpu.VMEM((T, D), jnp.bfloat16),
            pltpu.SemaphoreType.DMA((NSL,)),
            pltpu.SemaphoreType.DMA((NSL,)),
            pltpu.SemaphoreType.DMA((NRC + 1,)),
            pltpu.SemaphoreType.DMA((NRC + 1,)),
            pltpu.SemaphoreType.DMA((NRC,)),
            pltpu.SemaphoreType.DMA((NRC,)),
        ],
        compiler_params=pltpu.CompilerParams(
            vmem_limit_bytes=60 * 1024 * 1024,
            **({"collective_id": 0} if comm else {}),
        ),
    )(x, assign2, W1, W2)
